# Initial kernel scaffold; baseline (speedup 1.0000x reference)
import functools

import jax
import jax.numpy as jnp
from jax import lax
from jax.experimental import pallas as pl
from jax.experimental.pallas import tpu as pltpu
from jax.experimental.pallas import tpu_sc as plsc

M = 4194304
N = 419430
NC = 2
NS = 16
NT = NC * NS
CHUNK = 13120
PADN = NT * CHUNK
PAIRS = 2 * CHUNK
SLACK = 512
PACKW = PAIRS + SLACK
NB = 128
BSZ = M // NB
BIN_SHIFT = 15
BIN_MASK = BSZ - 1
BPT = NB // NT
CH = 512

_mesh = plsc.VectorSubcoreMesh(core_axis_name="c", subcore_axis_name="s")


@functools.partial(
    pl.kernel,
    mesh=_mesh,
    out_type=(
        jax.ShapeDtypeStruct((NT, PACKW), jnp.int32),
        jax.ShapeDtypeStruct((NT, PACKW), jnp.float32),
        jax.ShapeDtypeStruct((NT * NB,), jnp.int32),
    ),
    scratch_types=[
        pltpu.VMEM((CHUNK,), jnp.int32),
        pltpu.VMEM((CHUNK,), jnp.float32),
        pltpu.VMEM((CHUNK,), jnp.int32),
        pltpu.VMEM((CHUNK,), jnp.float32),
        pltpu.VMEM((NB,), jnp.int32),
        pltpu.VMEM((NB,), jnp.int32),
        pltpu.VMEM((PACKW,), jnp.int32),
        pltpu.VMEM((PACKW,), jnp.float32),
        pltpu.SemaphoreType.DMA,
    ],
)
def _bin_kernel(i1, v1, i2, v2, bi_out, bv_out, st_out,
                ti1, tv1, ti2, tv2, cnts, offs, pack_i, pack_v, sem):
    wid = lax.axis_index("c") * NS + lax.axis_index("s")
    base = wid * CHUNK
    cp1 = pltpu.async_copy(i1.at[pl.ds(base, CHUNK)], ti1, sem)
    cp2 = pltpu.async_copy(v1.at[pl.ds(base, CHUNK)], tv1, sem)
    cp3 = pltpu.async_copy(i2.at[pl.ds(base, CHUNK)], ti2, sem)
    cp4 = pltpu.async_copy(v2.at[pl.ds(base, CHUNK)], tv2, sem)

    zeros16 = jnp.zeros((16,), jnp.int32)
    ones16 = jnp.ones((16,), jnp.int32)
    neg16 = jnp.full((16,), -1, jnp.int32)
    for j in range(NB // 16):
        cnts[pl.ds(16 * j, 16)] = zeros16
    for j in range(SLACK // 16):
        pack_i[pl.ds(PAIRS + 16 * j, 16)] = neg16

    cp1.wait()
    cp2.wait()
    cp3.wait()
    cp4.wait()

    nv = CHUNK // 16

    def hist_body(q, carry):
        for ref in (ti1, ti2):
            x = ref[pl.ds(q * 16, 16)]
            bn = lax.shift_right_logical(x, BIN_SHIFT)
            plsc.addupdate_scatter(cnts, [bn], ones16)
        return carry

    lax.fori_loop(0, nv, hist_body, jnp.int32(0))

    def scan_body(j, carry):
        v = cnts[pl.ds(16 * j, 16)]
        cs = plsc.cumsum(v)
        excl = cs - v + carry
        cnts[pl.ds(16 * j, 16)] = excl
        offs[pl.ds(16 * j, 16)] = excl
        return carry + jnp.max(cs)

    lax.fori_loop(0, NB // 16, scan_body, jnp.int32(0))

    cnt0, _unused = plsc.scan_count(zeros16)
    rank_base = jnp.min(cnt0)

    def pack_body(q, carry):
        for iref, vref in ((ti1, tv1), (ti2, tv2)):
            x = iref[pl.ds(q * 16, 16)]
            v = vref[pl.ds(q * 16, 16)]
            bn = lax.shift_right_logical(x, BIN_SHIFT)
            cnt, _last = plsc.scan_count(bn)
            cur = plsc.load_gather(offs, [bn])
            slot = cur + (cnt - rank_base)
            plsc.store_scatter(pack_i, [slot], x)
            plsc.store_scatter(pack_v, [slot], v)
            plsc.addupdate_scatter(offs, [bn], ones16)
        return carry

    lax.fori_loop(0, nv, pack_body, jnp.int32(0))

    pltpu.sync_copy(pack_i, bi_out.at[wid])
    pltpu.sync_copy(pack_v, bv_out.at[wid])
    pltpu.sync_copy(cnts, st_out.at[pl.ds(wid * NB, NB)])


@functools.partial(
    pl.kernel,
    mesh=_mesh,
    out_type=jax.ShapeDtypeStruct((M,), jnp.float32),
    scratch_types=[
        pltpu.VMEM((NT * NB,), jnp.int32),
        pltpu.VMEM((BSZ,), jnp.float32),
        pltpu.VMEM((BSZ,), jnp.float32),
        pltpu.VMEM((CH,), jnp.int32),
        pltpu.VMEM((CH,), jnp.float32),
        pltpu.SemaphoreType.DMA,
    ],
)
def _merge_kernel(mem, bi, bv, st, out, starts, absmax, accum, sti, stv, sem):
    wid = lax.axis_index("c") * NS + lax.axis_index("s")
    pltpu.sync_copy(st, starts)

    for j in range(BPT):
        b = wid * BPT + j
        bin_base = b * BSZ
        zsrc = wid * (2 * BSZ)
        cz1 = pltpu.async_copy(mem.at[pl.ds(zsrc, BSZ)], absmax, sem)
        cz2 = pltpu.async_copy(mem.at[pl.ds(zsrc + BSZ, BSZ)], accum, sem)
        cz1.wait()
        cz2.wait()

        def sweep(phase):
            def t_body(t, carry):
                s = starts[t * NB + b]
                ei = jnp.minimum(t * NB + b + 1, NT * NB - 1)
                e = jnp.where(b == NB - 1, jnp.int32(PAIRS), starts[ei])
                a0 = s & jnp.int32(-8)
                nch = jnp.where(e > s, lax.shift_right_logical(e - a0 + CH - 1, 9), 0)

                def r_body(r, rcarry):
                    off = a0 + r * CH
                    c1 = pltpu.async_copy(bi.at[t, pl.ds(off, CH)], sti, sem)
                    c2 = pltpu.async_copy(bv.at[t, pl.ds(off, CH)], stv, sem)
                    c1.wait()
                    c2.wait()

                    def q_body(q, qcarry):
                        x = sti[pl.ds(q * 16, 16)]
                        v = stv[pl.ds(q * 16, 16)]
                        bn = lax.shift_right_logical(x, BIN_SHIFT)
                        msk = bn == b
                        ol = x & BIN_MASK
                        av = jnp.abs(v)
                        if phase == 0:
                            g = plsc.load_gather(absmax, [ol], mask=msk)
                            m = jnp.maximum(g, av)
                            plsc.store_scatter(absmax, [ol], m, mask=msk)
                            g2 = plsc.load_gather(absmax, [ol], mask=msk)
                            pend = jnp.where(msk & (av > g2), 1, 0)

                            def w_cond(p):
                                return jnp.max(p) > 0

                            def w_body(p):
                                pm = p > 0
                                gg = plsc.load_gather(absmax, [ol], mask=pm)
                                mm = jnp.maximum(gg, av)
                                plsc.store_scatter(absmax, [ol], mm, mask=pm)
                                gg2 = plsc.load_gather(absmax, [ol], mask=pm)
                                return jnp.where(pm & (av > gg2), 1, 0)

                            lax.while_loop(w_cond, w_body, pend)
                        else:
                            g = plsc.load_gather(absmax, [ol], mask=msk)
                            win = msk & (av == g)
                            plsc.addupdate_scatter(accum, [ol], v, mask=win)
                        return qcarry

                    lax.fori_loop(0, CH // 16, q_body, jnp.int32(0))
                    return rcarry

                lax.fori_loop(0, nch, r_body, jnp.int32(0))
                return carry

            lax.fori_loop(0, NT, t_body, jnp.int32(0))

        sweep(0)
        sweep(1)
        pltpu.sync_copy(accum, out.at[pl.ds(bin_base, BSZ)])


def kernel(mem, idx1, vals1, idx2, vals2):
    pad = PADN - N
    ip = jnp.zeros((pad,), jnp.int32)
    vp = jnp.zeros((pad,), jnp.float32)
    i1 = jnp.concatenate([idx1, ip])
    v1 = jnp.concatenate([vals1, vp])
    i2 = jnp.concatenate([idx2, ip])
    v2 = jnp.concatenate([vals2, vp])
    bi, bv, st = _bin_kernel(i1, v1, i2, v2)
    return _merge_kernel(mem, bi, bv, st)


# speedup vs baseline: 42.8845x; 42.8845x over previous
import functools

import jax
import jax.numpy as jnp
from jax import lax
from jax.experimental import pallas as pl
from jax.experimental.pallas import tpu as pltpu
from jax.experimental.pallas import tpu_sc as plsc

M = 4194304
N = 419430
NC = 2
NS = 16
NT = NC * NS
CHUNK = 13120
PADN = NT * CHUNK
PAIRS = 2 * CHUNK
SLACK = 512
PACKW = PAIRS + SLACK
NB = 128
BSZ = M // NB
BIN_SHIFT = 15
BIN_MASK = BSZ - 1
BPT = NB // NT
CH = 512

_mesh = plsc.VectorSubcoreMesh(core_axis_name="c", subcore_axis_name="s")


@functools.partial(
    pl.kernel,
    mesh=_mesh,
    compiler_params=pltpu.CompilerParams(needs_layout_passes=False),
    out_type=(
        jax.ShapeDtypeStruct((NT * PACKW,), jnp.int32),
        jax.ShapeDtypeStruct((NT * PACKW,), jnp.float32),
        jax.ShapeDtypeStruct((NT * NB,), jnp.int32),
    ),
    scratch_types=[
        pltpu.VMEM((CHUNK,), jnp.int32),
        pltpu.VMEM((CHUNK,), jnp.float32),
        pltpu.VMEM((CHUNK,), jnp.int32),
        pltpu.VMEM((CHUNK,), jnp.float32),
        pltpu.VMEM((NB,), jnp.int32),
        pltpu.VMEM((NB,), jnp.int32),
        pltpu.VMEM((PACKW,), jnp.int32),
        pltpu.VMEM((PACKW,), jnp.float32),
        pltpu.SemaphoreType.DMA,
    ],
)
def _bin_kernel(i1, v1, i2, v2, bi_out, bv_out, st_out,
                ti1, tv1, ti2, tv2, cnts, offs, pack_i, pack_v, sem):
    wid = lax.axis_index("c") * NS + lax.axis_index("s")
    base = pl.multiple_of(wid * CHUNK, 8)
    cp1 = pltpu.async_copy(i1.at[pl.ds(base, CHUNK)], ti1, sem)
    cp2 = pltpu.async_copy(v1.at[pl.ds(base, CHUNK)], tv1, sem)
    cp3 = pltpu.async_copy(i2.at[pl.ds(base, CHUNK)], ti2, sem)
    cp4 = pltpu.async_copy(v2.at[pl.ds(base, CHUNK)], tv2, sem)

    zeros16 = jnp.zeros((16,), jnp.int32)
    ones16 = jnp.ones((16,), jnp.int32)
    neg16 = jnp.full((16,), -1, jnp.int32)
    for j in range(NB // 16):
        cnts[pl.ds(16 * j, 16)] = zeros16
    for j in range(SLACK // 16):
        pack_i[pl.ds(PAIRS + 16 * j, 16)] = neg16

    cp1.wait()
    cp2.wait()
    cp3.wait()
    cp4.wait()

    nv = CHUNK // 16

    def hist_body(q, carry):
        for ref in (ti1, ti2):
            x = ref[pl.ds(q * 16, 16)]
            bn = lax.shift_right_logical(x, BIN_SHIFT)
            plsc.addupdate_scatter(cnts, [bn], ones16)
        return carry

    lax.fori_loop(0, nv, hist_body, jnp.int32(0))

    def scan_body(j, carry):
        v = cnts[pl.ds(16 * j, 16)]
        cs = plsc.cumsum(v)
        excl = cs - v + carry
        cnts[pl.ds(16 * j, 16)] = excl
        offs[pl.ds(16 * j, 16)] = excl
        return carry + jnp.max(cs)

    lax.fori_loop(0, NB // 16, scan_body, jnp.int32(0))

    cnt0, _unused = plsc.scan_count(zeros16)
    rank_base = jnp.min(cnt0)

    def pack_body(q, carry):
        for iref, vref in ((ti1, tv1), (ti2, tv2)):
            x = iref[pl.ds(q * 16, 16)]
            v = vref[pl.ds(q * 16, 16)]
            bn = lax.shift_right_logical(x, BIN_SHIFT)
            cnt, _last = plsc.scan_count(bn)
            cur = plsc.load_gather(offs, [bn])
            slot = cur + (cnt - rank_base)
            plsc.store_scatter(pack_i, [slot], x)
            plsc.store_scatter(pack_v, [slot], v)
            plsc.addupdate_scatter(offs, [bn], ones16)
        return carry

    lax.fori_loop(0, nv, pack_body, jnp.int32(0))

    pltpu.sync_copy(pack_i, bi_out.at[pl.ds(pl.multiple_of(wid * PACKW, 8), PACKW)])
    pltpu.sync_copy(pack_v, bv_out.at[pl.ds(pl.multiple_of(wid * PACKW, 8), PACKW)])
    pltpu.sync_copy(cnts, st_out.at[pl.ds(pl.multiple_of(wid * NB, 8), NB)])


@functools.partial(
    pl.kernel,
    mesh=_mesh,
    compiler_params=pltpu.CompilerParams(needs_layout_passes=False),
    out_type=jax.ShapeDtypeStruct((M,), jnp.float32),
    scratch_types=[
        pltpu.VMEM((NT * NB + 16,), jnp.int32),
        pltpu.VMEM((BSZ,), jnp.float32),
        pltpu.VMEM((BSZ,), jnp.float32),
        pltpu.VMEM((CH,), jnp.int32),
        pltpu.VMEM((CH,), jnp.float32),
        pltpu.SemaphoreType.DMA,
    ],
)
def _merge_kernel(mem, bi, bv, st, out, starts, absmax, accum, sti, stv, sem):
    wid = lax.axis_index("c") * NS + lax.axis_index("s")
    pltpu.sync_copy(st, starts.at[pl.ds(0, NT * NB)])

    def sread(idx):
        return jnp.max(plsc.load_gather(starts, [lax.broadcast(idx, (16,))]))

    for j in range(BPT):
        b = wid * BPT + j
        bin_base = pl.multiple_of(b * BSZ, 8)
        zsrc = pl.multiple_of(wid * (2 * BSZ), 8)
        cz1 = pltpu.async_copy(mem.at[pl.ds(zsrc, BSZ)], absmax, sem)
        cz2 = pltpu.async_copy(mem.at[pl.ds(zsrc + BSZ, BSZ)], accum, sem)
        cz1.wait()
        cz2.wait()

        def sweep(phase):
            def t_body(t, carry):
                s = sread(t * NB + b)
                ei = jnp.minimum(t * NB + b + 1, NT * NB - 1)
                e = jnp.where(b == NB - 1, jnp.int32(PAIRS), sread(ei))
                a0 = pl.multiple_of(s & jnp.int32(-8), 8)
                nch = jnp.where(e > s, lax.shift_right_logical(e - a0 + CH - 1, 9), 0)

                def r_body(r, rcarry):
                    off = pl.multiple_of(a0 + r * CH, 8)
                    src = pl.multiple_of(t * PACKW + off, 8)
                    c1 = pltpu.async_copy(bi.at[pl.ds(src, CH)], sti, sem)
                    c2 = pltpu.async_copy(bv.at[pl.ds(src, CH)], stv, sem)
                    c1.wait()
                    c2.wait()

                    def q_body(q, qcarry):
                        x = sti[pl.ds(q * 16, 16)]
                        v = stv[pl.ds(q * 16, 16)]
                        bn = lax.shift_right_logical(x, BIN_SHIFT)
                        msk = bn == b
                        ol = x & BIN_MASK
                        av = jnp.abs(v)
                        if phase == 0:
                            g = plsc.load_gather(absmax, [ol], mask=msk)
                            m = jnp.maximum(g, av)
                            plsc.store_scatter(absmax, [ol], m, mask=msk)
                            g2 = plsc.load_gather(absmax, [ol], mask=msk)
                            pend = jnp.where(msk & (av > g2), 1, 0)

                            def w_cond(p):
                                return jnp.max(p) > 0

                            def w_body(p):
                                pm = p > 0
                                gg = plsc.load_gather(absmax, [ol], mask=pm)
                                mm = jnp.maximum(gg, av)
                                plsc.store_scatter(absmax, [ol], mm, mask=pm)
                                gg2 = plsc.load_gather(absmax, [ol], mask=pm)
                                return jnp.where(pm & (av > gg2), 1, 0)

                            lax.while_loop(w_cond, w_body, pend)
                        else:
                            g = plsc.load_gather(absmax, [ol], mask=msk)
                            win = msk & (av == g)
                            plsc.addupdate_scatter(accum, [ol], v, mask=win)
                        return qcarry

                    lax.fori_loop(0, CH // 16, q_body, jnp.int32(0))
                    return rcarry

                lax.fori_loop(0, nch, r_body, jnp.int32(0))
                return carry

            lax.fori_loop(0, NT, t_body, jnp.int32(0))

        sweep(0)
        sweep(1)
        pltpu.sync_copy(accum, out.at[pl.ds(bin_base, BSZ)])


def kernel(mem, idx1, vals1, idx2, vals2):
    pad = PADN - N
    ip = jnp.zeros((pad,), jnp.int32)
    vp = jnp.zeros((pad,), jnp.float32)
    i1 = jnp.concatenate([idx1, ip])
    v1 = jnp.concatenate([vals1, vp])
    i2 = jnp.concatenate([idx2, ip])
    v2 = jnp.concatenate([vals2, vp])
    bi, bv, st = _bin_kernel(i1, v1, i2, v2)
    return _merge_kernel(mem, bi, bv, st)


# speedup vs baseline: 57.2999x; 1.3361x over previous
import functools

import jax
import jax.numpy as jnp
from jax import lax
from jax.experimental import pallas as pl
from jax.experimental.pallas import tpu as pltpu
from jax.experimental.pallas import tpu_sc as plsc

M = 4194304
N = 419430
NC = 2
NS = 16
NT = NC * NS
CHUNK = 13120
PADN = NT * CHUNK
PAIRS = 2 * CHUNK
SLACK = 512
PACKW = PAIRS + SLACK
NB = 128
BSZ = M // NB
BIN_SHIFT = 15
BIN_MASK = BSZ - 1
BPT = NB // NT
CH = 512

_mesh = plsc.VectorSubcoreMesh(core_axis_name="c", subcore_axis_name="s")


@functools.partial(
    pl.kernel,
    mesh=_mesh,
    compiler_params=pltpu.CompilerParams(needs_layout_passes=False),
    out_type=(
        jax.ShapeDtypeStruct((NT * PACKW,), jnp.int32),
        jax.ShapeDtypeStruct((NT * PACKW,), jnp.float32),
        jax.ShapeDtypeStruct((NT * NB,), jnp.int32),
    ),
    scratch_types=[
        pltpu.VMEM((CHUNK,), jnp.int32),
        pltpu.VMEM((CHUNK,), jnp.float32),
        pltpu.VMEM((CHUNK,), jnp.int32),
        pltpu.VMEM((CHUNK,), jnp.float32),
        pltpu.VMEM((NB,), jnp.int32),
        pltpu.VMEM((NB,), jnp.int32),
        pltpu.VMEM((PACKW,), jnp.int32),
        pltpu.VMEM((PACKW,), jnp.float32),
        pltpu.SemaphoreType.DMA,
    ],
)
def _bin_kernel(i1, v1, i2, v2, bi_out, bv_out, st_out,
                ti1, tv1, ti2, tv2, cnts, offs, pack_i, pack_v, sem):
    wid = lax.axis_index("c") * NS + lax.axis_index("s")
    base = pl.multiple_of(wid * CHUNK, 8)
    cp1 = pltpu.async_copy(i1.at[pl.ds(base, CHUNK)], ti1, sem)
    cp2 = pltpu.async_copy(v1.at[pl.ds(base, CHUNK)], tv1, sem)
    cp3 = pltpu.async_copy(i2.at[pl.ds(base, CHUNK)], ti2, sem)
    cp4 = pltpu.async_copy(v2.at[pl.ds(base, CHUNK)], tv2, sem)

    zeros16 = jnp.zeros((16,), jnp.int32)
    ones16 = jnp.ones((16,), jnp.int32)
    neg16 = jnp.full((16,), -1, jnp.int32)
    for j in range(NB // 16):
        cnts[pl.ds(16 * j, 16)] = zeros16
    for j in range(SLACK // 16):
        pack_i[pl.ds(PAIRS + 16 * j, 16)] = neg16

    cp1.wait()
    cp2.wait()
    cp3.wait()
    cp4.wait()

    nv = CHUNK // 16

    def hist_body(q, carry):
        for ref in (ti1, ti2):
            x = ref[pl.ds(q * 16, 16)]
            bn = lax.shift_right_logical(x, BIN_SHIFT)
            plsc.addupdate_scatter(cnts, [bn], ones16)
        return carry

    lax.fori_loop(0, nv, hist_body, jnp.int32(0))

    def scan_body(j, carry):
        v = cnts[pl.ds(16 * j, 16)]
        cs = plsc.cumsum(v)
        excl = cs - v + carry
        cnts[pl.ds(16 * j, 16)] = excl
        offs[pl.ds(16 * j, 16)] = excl
        return carry + jnp.max(cs)

    lax.fori_loop(0, NB // 16, scan_body, jnp.int32(0))

    cnt0, _unused = plsc.scan_count(zeros16)
    rank_base = jnp.min(cnt0)

    def pack_body(q, carry):
        for iref, vref in ((ti1, tv1), (ti2, tv2)):
            x = iref[pl.ds(q * 16, 16)]
            v = vref[pl.ds(q * 16, 16)]
            bn = lax.shift_right_logical(x, BIN_SHIFT)
            cnt, _last = plsc.scan_count(bn)
            cur = plsc.load_gather(offs, [bn])
            slot = cur + (cnt - rank_base)
            plsc.store_scatter(pack_i, [slot], x)
            plsc.store_scatter(pack_v, [slot], v)
            plsc.addupdate_scatter(offs, [bn], ones16)
        return carry

    lax.fori_loop(0, nv, pack_body, jnp.int32(0))

    pltpu.sync_copy(pack_i, bi_out.at[pl.ds(pl.multiple_of(wid * PACKW, 8), PACKW)])
    pltpu.sync_copy(pack_v, bv_out.at[pl.ds(pl.multiple_of(wid * PACKW, 8), PACKW)])
    pltpu.sync_copy(cnts, st_out.at[pl.ds(pl.multiple_of(wid * NB, 8), NB)])


@functools.partial(
    pl.kernel,
    mesh=_mesh,
    compiler_params=pltpu.CompilerParams(needs_layout_passes=False),
    out_type=jax.ShapeDtypeStruct((M,), jnp.float32),
    scratch_types=[
        pltpu.VMEM((NT * NB + 16,), jnp.int32),
        pltpu.VMEM((BSZ,), jnp.float32),
        pltpu.VMEM((BSZ,), jnp.float32),
        pltpu.VMEM((CH,), jnp.int32),
        pltpu.VMEM((CH,), jnp.float32),
        pltpu.VMEM((CH,), jnp.int32),
        pltpu.VMEM((CH,), jnp.float32),
        pltpu.VMEM((CH,), jnp.int32),
        pltpu.VMEM((CH,), jnp.float32),
        pltpu.VMEM((NT,), jnp.int32),
        pltpu.VMEM((NT,), jnp.int32),
        pltpu.VMEM((NT,), jnp.int32),
        pltpu.VMEM((NT,), jnp.int32),
        pltpu.SemaphoreType.DMA,
        pltpu.SemaphoreType.DMA,
        pltpu.SemaphoreType.DMA,
        pltpu.SemaphoreType.DMA,
        pltpu.SemaphoreType.DMA,
    ],
)
def _merge_kernel(mem, bi, bv, st, out, starts, absmax, accum,
                  b0i, b0v, b1i, b1v, bci, bcv,
                  seg_s, seg_e, seg_a0, seg_n,
                  sem0, sem1, semc, semza, semzb):
    wid = lax.axis_index("c") * NS + lax.axis_index("s")
    pltpu.sync_copy(st, starts.at[pl.ds(0, NT * NB)])
    iota16 = lax.broadcasted_iota(jnp.int32, (16,), 0)

    def sget(ref, i):
        return jnp.max(plsc.load_gather(ref, [lax.broadcast(i, (16,))]))

    def issue(tt, dbi, dbv, sem):
        a0 = sget(seg_a0, tt)
        src = pl.multiple_of(tt * PACKW + a0, 8)
        pltpu.async_copy(bi.at[pl.ds(src, CH)], dbi, sem)
        pltpu.async_copy(bv.at[pl.ds(src, CH)], dbv, sem)

    def waitb(dbi, dbv, sem):
        pltpu.make_async_copy(bi.at[pl.ds(0, CH)], dbi, sem).wait()
        pltpu.make_async_copy(bv.at[pl.ds(0, CH)], dbv, sem).wait()

    def jbody(j, jcarry):
        b = wid * BPT + j
        bin_base = pl.multiple_of(b * BSZ, 8)
        zsrc = pl.multiple_of(wid * (2 * BSZ), 8)
        cza = pltpu.async_copy(mem.at[pl.ds(zsrc, BSZ)], absmax, semza)
        czb = pltpu.async_copy(mem.at[pl.ds(zsrc + BSZ, BSZ)], accum, semzb)

        for g in range(NT // 16):
            tv = g * 16 + iota16
            idxv = tv * NB + b
            sv = plsc.load_gather(starts, [idxv])
            ev = plsc.load_gather(starts, [idxv + 1])
            ev = jnp.where(b == NB - 1, jnp.int32(PAIRS), ev)
            a0v = sv & jnp.int32(-8)
            nv_ = jnp.where(
                ev > sv,
                lax.shift_right_logical(ev - a0v + jnp.int32(CH - 1), 9),
                0,
            )
            seg_s[pl.ds(g * 16, 16)] = sv
            seg_e[pl.ds(g * 16, 16)] = ev
            seg_a0[pl.ds(g * 16, 16)] = a0v
            seg_n[pl.ds(g * 16, 16)] = nv_

        def qloop(phase, dbi, dbv, q_lo, q_hi):
            def q_body(q, qc):
                x = dbi[pl.ds(q * 16, 16)]
                v = dbv[pl.ds(q * 16, 16)]
                bn = lax.shift_right_logical(x, BIN_SHIFT)
                msk = bn == b
                ol = x & BIN_MASK
                av = jnp.abs(v)
                if phase == 0:
                    g_ = plsc.load_gather(absmax, [ol], mask=msk)
                    m = jnp.maximum(g_, av)
                    plsc.store_scatter(absmax, [ol], m, mask=msk)
                    g2 = plsc.load_gather(absmax, [ol], mask=msk)
                    pend = jnp.where(msk & (av > g2), 1, 0)

                    def w_cond(pc):
                        return jnp.max(pc) > 0

                    def w_body(pc):
                        pm = pc > 0
                        gg = plsc.load_gather(absmax, [ol], mask=pm)
                        mm = jnp.maximum(gg, av)
                        plsc.store_scatter(absmax, [ol], mm, mask=pm)
                        gg2 = plsc.load_gather(absmax, [ol], mask=pm)
                        return jnp.where(pm & (av > gg2), 1, 0)

                    lax.while_loop(w_cond, w_body, pend)
                else:
                    g_ = plsc.load_gather(absmax, [ol], mask=msk)
                    win = msk & (av == g_)
                    plsc.addupdate_scatter(accum, [ol], v, mask=win)
                return qc

            lax.fori_loop(q_lo, q_hi, q_body, jnp.int32(0))

        def process(phase, tt, dbi, dbv):
            s = sget(seg_s, tt)
            e = sget(seg_e, tt)
            a0 = sget(seg_a0, tt)
            n = sget(seg_n, tt)
            q_lo = lax.shift_right_logical(s - a0, 4)
            q_hi = lax.shift_right_logical(jnp.minimum(e, a0 + CH) - a0 + 15, 4)
            qloop(phase, dbi, dbv, q_lo, q_hi)

            def r_body(r, rc):
                off = pl.multiple_of(a0 + r * CH, 8)
                src = pl.multiple_of(tt * PACKW + off, 8)
                c1 = pltpu.async_copy(bi.at[pl.ds(src, CH)], bci, semc)
                c2 = pltpu.async_copy(bv.at[pl.ds(src, CH)], bcv, semc)
                c1.wait()
                c2.wait()
                qh = lax.shift_right_logical(jnp.minimum(e, off + CH) - off + 15, 4)
                qloop(phase, bci, bcv, jnp.int32(0), qh)
                return rc

            lax.fori_loop(1, n, r_body, jnp.int32(0))

        def sweep(phase):
            def k_body(k, kc):
                issue(k, b0i, b0v, sem0)
                waitb(b0i, b0v, sem0)
                process(phase, k, b0i, b0v)
                return kc

            lax.fori_loop(0, NT, k_body, jnp.int32(0))

        cza.wait()
        sweep(0)
        czb.wait()
        sweep(1)
        pltpu.sync_copy(accum, out.at[pl.ds(bin_base, BSZ)])
        return jcarry

    lax.fori_loop(0, BPT, jbody, jnp.int32(0))


def kernel(mem, idx1, vals1, idx2, vals2):
    pad = PADN - N
    ip = jnp.zeros((pad,), jnp.int32)
    vp = jnp.zeros((pad,), jnp.float32)
    i1 = jnp.concatenate([idx1, ip])
    v1 = jnp.concatenate([vals1, vp])
    i2 = jnp.concatenate([idx2, ip])
    v2 = jnp.concatenate([vals2, vp])
    bi, bv, st = _bin_kernel(i1, v1, i2, v2)
    return _merge_kernel(mem, bi, bv, st)


# speedup vs baseline: 80.1207x; 1.3983x over previous
import functools

import jax
import jax.numpy as jnp
from jax import lax
from jax.experimental import pallas as pl
from jax.experimental.pallas import tpu as pltpu
from jax.experimental.pallas import tpu_sc as plsc

M = 4194304
N = 419430
NC = 2
NS = 16
NT = NC * NS
CHUNK = 13120
PADN = NT * CHUNK
PAIRS = 2 * CHUNK
SLACK = 512
PACKW = PAIRS + SLACK
NB = 128
BSZ = M // NB
BIN_SHIFT = 15
BIN_MASK = BSZ - 1
BPT = NB // NT
CH = 512

_mesh = plsc.VectorSubcoreMesh(core_axis_name="c", subcore_axis_name="s")


@functools.partial(
    pl.kernel,
    mesh=_mesh,
    compiler_params=pltpu.CompilerParams(needs_layout_passes=False),
    out_type=(
        jax.ShapeDtypeStruct((NT * PACKW,), jnp.int32),
        jax.ShapeDtypeStruct((NT * PACKW,), jnp.float32),
        jax.ShapeDtypeStruct((NT * NB,), jnp.int32),
    ),
    scratch_types=[
        pltpu.VMEM((CHUNK,), jnp.int32),
        pltpu.VMEM((CHUNK,), jnp.float32),
        pltpu.VMEM((CHUNK,), jnp.int32),
        pltpu.VMEM((CHUNK,), jnp.float32),
        pltpu.VMEM((NB,), jnp.int32),
        pltpu.VMEM((NB,), jnp.int32),
        pltpu.VMEM((PACKW,), jnp.int32),
        pltpu.VMEM((PACKW,), jnp.float32),
        pltpu.SemaphoreType.DMA,
    ],
)
def _bin_kernel(i1, v1, i2, v2, bi_out, bv_out, st_out,
                ti1, tv1, ti2, tv2, cnts, offs, pack_i, pack_v, sem):
    wid = lax.axis_index("c") * NS + lax.axis_index("s")
    base = pl.multiple_of(wid * CHUNK, 8)
    cp1 = pltpu.async_copy(i1.at[pl.ds(base, CHUNK)], ti1, sem)
    cp2 = pltpu.async_copy(v1.at[pl.ds(base, CHUNK)], tv1, sem)
    cp3 = pltpu.async_copy(i2.at[pl.ds(base, CHUNK)], ti2, sem)
    cp4 = pltpu.async_copy(v2.at[pl.ds(base, CHUNK)], tv2, sem)

    zeros16 = jnp.zeros((16,), jnp.int32)
    ones16 = jnp.ones((16,), jnp.int32)
    neg16 = jnp.full((16,), -1, jnp.int32)
    for j in range(NB // 16):
        cnts[pl.ds(16 * j, 16)] = zeros16
    for j in range(SLACK // 16):
        pack_i[pl.ds(PAIRS + 16 * j, 16)] = neg16

    cp1.wait()
    cp2.wait()
    cp3.wait()
    cp4.wait()

    nv = CHUNK // 16

    def hist_body(q, carry):
        for ref in (ti1, ti2):
            x = ref[pl.ds(q * 16, 16)]
            bn = lax.shift_right_logical(x, BIN_SHIFT)
            plsc.addupdate_scatter(cnts, [bn], ones16)
        return carry

    lax.fori_loop(0, nv, hist_body, jnp.int32(0))

    def scan_body(j, carry):
        v = cnts[pl.ds(16 * j, 16)]
        cs = plsc.cumsum(v)
        excl = cs - v + carry
        cnts[pl.ds(16 * j, 16)] = excl
        offs[pl.ds(16 * j, 16)] = excl
        return carry + jnp.max(cs)

    lax.fori_loop(0, NB // 16, scan_body, jnp.int32(0))

    cnt0, _unused = plsc.scan_count(zeros16)
    rank_base = jnp.min(cnt0)

    def pack_body(q, carry):
        for iref, vref in ((ti1, tv1), (ti2, tv2)):
            x = iref[pl.ds(q * 16, 16)]
            v = vref[pl.ds(q * 16, 16)]
            bn = lax.shift_right_logical(x, BIN_SHIFT)
            cnt, _last = plsc.scan_count(bn)
            cur = plsc.load_gather(offs, [bn])
            slot = cur + (cnt - rank_base)
            plsc.store_scatter(pack_i, [slot], x)
            plsc.store_scatter(pack_v, [slot], v)
            plsc.addupdate_scatter(offs, [bn], ones16)
        return carry

    lax.fori_loop(0, nv, pack_body, jnp.int32(0))

    pltpu.sync_copy(pack_i, bi_out.at[pl.ds(pl.multiple_of(wid * PACKW, 8), PACKW)])
    pltpu.sync_copy(pack_v, bv_out.at[pl.ds(pl.multiple_of(wid * PACKW, 8), PACKW)])
    pltpu.sync_copy(cnts, st_out.at[pl.ds(pl.multiple_of(wid * NB, 8), NB)])


@functools.partial(
    pl.kernel,
    mesh=_mesh,
    compiler_params=pltpu.CompilerParams(needs_layout_passes=False),
    out_type=jax.ShapeDtypeStruct((M,), jnp.float32),
    scratch_types=[
        pltpu.VMEM((NT * NB + 16,), jnp.int32),
        pltpu.VMEM((BSZ,), jnp.float32),
        pltpu.VMEM((BSZ,), jnp.float32),
        pltpu.VMEM((CH,), jnp.int32),
        pltpu.VMEM((CH,), jnp.float32),
        pltpu.VMEM((CH,), jnp.int32),
        pltpu.VMEM((CH,), jnp.float32),
        pltpu.VMEM((CH,), jnp.int32),
        pltpu.VMEM((CH,), jnp.float32),
        pltpu.VMEM((CH,), jnp.int32),
        pltpu.VMEM((CH,), jnp.float32),
        pltpu.VMEM((CH,), jnp.int32),
        pltpu.VMEM((CH,), jnp.float32),
        pltpu.VMEM((NT,), jnp.int32),
        pltpu.VMEM((NT,), jnp.int32),
        pltpu.VMEM((NT,), jnp.int32),
        pltpu.VMEM((NT,), jnp.int32),
        pltpu.SemaphoreType.DMA,
        pltpu.SemaphoreType.DMA,
        pltpu.SemaphoreType.DMA,
        pltpu.SemaphoreType.DMA,
        pltpu.SemaphoreType.DMA,
    ],
)
def _merge_kernel(mem, bi, bv, st, out, starts, absmax, accum,
                  b0i, b0v, b1i, b1v, b2i, b2v, b3i, b3v, bci, bcv,
                  seg_s, seg_e, seg_a0, seg_n,
                  sem0, sem1, semc, semza, semzb):
    wid = lax.axis_index("c") * NS + lax.axis_index("s")
    pltpu.sync_copy(st, starts.at[pl.ds(0, NT * NB)])
    iota16 = lax.broadcasted_iota(jnp.int32, (16,), 0)

    def sget(ref, i):
        return jnp.max(plsc.load_gather(ref, [lax.broadcast(i, (16,))]))

    def issue(tt, dbi, dbv, sem):
        a0 = sget(seg_a0, tt)
        src = pl.multiple_of(tt * PACKW + a0, 8)
        pltpu.async_copy(bi.at[pl.ds(src, CH)], dbi, sem)
        pltpu.async_copy(bv.at[pl.ds(src, CH)], dbv, sem)

    def waitb(dbi, dbv, sem):
        pltpu.make_async_copy(bi.at[pl.ds(0, CH)], dbi, sem).wait()
        pltpu.make_async_copy(bv.at[pl.ds(0, CH)], dbv, sem).wait()

    def jbody(j, jcarry):
        b = wid * BPT + j
        bin_base = pl.multiple_of(b * BSZ, 8)
        zsrc = pl.multiple_of(wid * (2 * BSZ), 8)
        cza = pltpu.async_copy(mem.at[pl.ds(zsrc, BSZ)], absmax, semza)
        czb = pltpu.async_copy(mem.at[pl.ds(zsrc + BSZ, BSZ)], accum, semzb)

        for g in range(NT // 16):
            tv = g * 16 + iota16
            idxv = tv * NB + b
            sv = plsc.load_gather(starts, [idxv])
            ev = plsc.load_gather(starts, [idxv + 1])
            ev = jnp.where(b == NB - 1, jnp.int32(PAIRS), ev)
            a0v = sv & jnp.int32(-8)
            nv_ = jnp.where(
                ev > sv,
                lax.shift_right_logical(ev - a0v + jnp.int32(CH - 1), 9),
                0,
            )
            seg_s[pl.ds(g * 16, 16)] = sv
            seg_e[pl.ds(g * 16, 16)] = ev
            seg_a0[pl.ds(g * 16, 16)] = a0v
            seg_n[pl.ds(g * 16, 16)] = nv_

        def qloop(phase, dbi, dbv, q_lo, q_hi):
            def q_body(q, qc):
                x = dbi[pl.ds(q * 16, 16)]
                v = dbv[pl.ds(q * 16, 16)]
                bn = lax.shift_right_logical(x, BIN_SHIFT)
                msk = bn == b
                ol = x & BIN_MASK
                av = jnp.abs(v)
                if phase == 0:
                    g_ = plsc.load_gather(absmax, [ol], mask=msk)
                    m = jnp.maximum(g_, av)
                    plsc.store_scatter(absmax, [ol], m, mask=msk)
                    g2 = plsc.load_gather(absmax, [ol], mask=msk)
                    pend = jnp.where(msk & (av > g2), 1, 0)

                    def w_cond(pc):
                        return jnp.max(pc) > 0

                    def w_body(pc):
                        pm = pc > 0
                        gg = plsc.load_gather(absmax, [ol], mask=pm)
                        mm = jnp.maximum(gg, av)
                        plsc.store_scatter(absmax, [ol], mm, mask=pm)
                        gg2 = plsc.load_gather(absmax, [ol], mask=pm)
                        return jnp.where(pm & (av > gg2), 1, 0)

                    lax.while_loop(w_cond, w_body, pend)
                else:
                    g_ = plsc.load_gather(absmax, [ol], mask=msk)
                    win = msk & (av == g_)
                    plsc.addupdate_scatter(accum, [ol], v, mask=win)
                return qc

            lax.fori_loop(q_lo, q_hi, q_body, jnp.int32(0))

        def process(phase, tt, dbi, dbv):
            s = sget(seg_s, tt)
            e = sget(seg_e, tt)
            a0 = sget(seg_a0, tt)
            n = sget(seg_n, tt)
            q_lo = lax.shift_right_logical(s - a0, 4)
            q_hi = lax.shift_right_logical(jnp.minimum(e, a0 + CH) - a0 + 15, 4)
            qloop(phase, dbi, dbv, q_lo, q_hi)

            def r_body(r, rc):
                off = pl.multiple_of(a0 + r * CH, 8)
                src = pl.multiple_of(tt * PACKW + off, 8)
                c1 = pltpu.async_copy(bi.at[pl.ds(src, CH)], bci, semc)
                c2 = pltpu.async_copy(bv.at[pl.ds(src, CH)], bcv, semc)
                c1.wait()
                c2.wait()
                qh = lax.shift_right_logical(jnp.minimum(e, off + CH) - off + 15, 4)
                qloop(phase, bci, bcv, jnp.int32(0), qh)
                return rc

            lax.fori_loop(1, n, r_body, jnp.int32(0))

        def sweep(phase):
            bufs = ((b0i, b0v), (b1i, b1v), (b2i, b2v), (b3i, b3v))

            def k_body(k, kc):
                t0 = 4 * k
                for u, (dbi, dbv) in enumerate(bufs):
                    issue(t0 + u, dbi, dbv, sem0)
                for dbi, dbv in bufs:
                    waitb(dbi, dbv, sem0)
                for u, (dbi, dbv) in enumerate(bufs):
                    process(phase, t0 + u, dbi, dbv)
                return kc

            lax.fori_loop(0, NT // 4, k_body, jnp.int32(0))

        cza.wait()
        sweep(0)
        czb.wait()
        sweep(1)
        pltpu.sync_copy(accum, out.at[pl.ds(bin_base, BSZ)])
        return jcarry

    lax.fori_loop(0, BPT, jbody, jnp.int32(0))


def kernel(mem, idx1, vals1, idx2, vals2):
    pad = PADN - N
    ip = jnp.zeros((pad,), jnp.int32)
    vp = jnp.zeros((pad,), jnp.float32)
    i1 = jnp.concatenate([idx1, ip])
    v1 = jnp.concatenate([vals1, vp])
    i2 = jnp.concatenate([idx2, ip])
    v2 = jnp.concatenate([vals2, vp])
    bi, bv, st = _bin_kernel(i1, v1, i2, v2)
    return _merge_kernel(mem, bi, bv, st)


# speedup vs baseline: 81.8603x; 1.0217x over previous
import functools

import jax
import jax.numpy as jnp
from jax import lax
from jax.experimental import pallas as pl
from jax.experimental.pallas import tpu as pltpu
from jax.experimental.pallas import tpu_sc as plsc

M = 4194304
N = 419430
NC = 2
NS = 16
NT = NC * NS
CHUNK = 13120
PADN = NT * CHUNK
PAIRS = 2 * CHUNK
SLACK = 512
PACKW = PAIRS + SLACK
NB = 128
BSZ = M // NB
BIN_SHIFT = 15
BIN_MASK = BSZ - 1
BPT = NB // NT
CH = 256
CHS = CH.bit_length() - 1

_mesh = plsc.VectorSubcoreMesh(core_axis_name="c", subcore_axis_name="s")


@functools.partial(
    pl.kernel,
    mesh=_mesh,
    compiler_params=pltpu.CompilerParams(needs_layout_passes=False),
    out_type=(
        jax.ShapeDtypeStruct((NT * PACKW,), jnp.int32),
        jax.ShapeDtypeStruct((NT * PACKW,), jnp.float32),
        jax.ShapeDtypeStruct((NT * NB,), jnp.int32),
    ),
    scratch_types=[
        pltpu.VMEM((CHUNK,), jnp.int32),
        pltpu.VMEM((CHUNK,), jnp.float32),
        pltpu.VMEM((CHUNK,), jnp.int32),
        pltpu.VMEM((CHUNK,), jnp.float32),
        pltpu.VMEM((NB,), jnp.int32),
        pltpu.VMEM((NB,), jnp.int32),
        pltpu.VMEM((PACKW,), jnp.int32),
        pltpu.VMEM((PACKW,), jnp.float32),
        pltpu.SemaphoreType.DMA,
    ],
)
def _bin_kernel(i1, v1, i2, v2, bi_out, bv_out, st_out,
                ti1, tv1, ti2, tv2, cnts, offs, pack_i, pack_v, sem):
    wid = lax.axis_index("c") * NS + lax.axis_index("s")
    base = pl.multiple_of(wid * CHUNK, 8)
    cp1 = pltpu.async_copy(i1.at[pl.ds(base, CHUNK)], ti1, sem)
    cp2 = pltpu.async_copy(v1.at[pl.ds(base, CHUNK)], tv1, sem)
    cp3 = pltpu.async_copy(i2.at[pl.ds(base, CHUNK)], ti2, sem)
    cp4 = pltpu.async_copy(v2.at[pl.ds(base, CHUNK)], tv2, sem)

    zeros16 = jnp.zeros((16,), jnp.int32)
    ones16 = jnp.ones((16,), jnp.int32)
    neg16 = jnp.full((16,), -1, jnp.int32)
    for j in range(NB // 16):
        cnts[pl.ds(16 * j, 16)] = zeros16
    for j in range(SLACK // 16):
        pack_i[pl.ds(PAIRS + 16 * j, 16)] = neg16

    cp1.wait()
    cp2.wait()
    cp3.wait()
    cp4.wait()

    nv = CHUNK // 16

    def hist_body(q, carry):
        for ref in (ti1, ti2):
            x = ref[pl.ds(q * 16, 16)]
            bn = lax.shift_right_logical(x, BIN_SHIFT)
            plsc.addupdate_scatter(cnts, [bn], ones16)
        return carry

    lax.fori_loop(0, nv, hist_body, jnp.int32(0), unroll=4)

    def scan_body(j, carry):
        v = cnts[pl.ds(16 * j, 16)]
        cs = plsc.cumsum(v)
        excl = cs - v + carry
        cnts[pl.ds(16 * j, 16)] = excl
        offs[pl.ds(16 * j, 16)] = excl
        return carry + jnp.max(cs)

    lax.fori_loop(0, NB // 16, scan_body, jnp.int32(0))

    cnt0, _unused = plsc.scan_count(zeros16)
    rank_base = jnp.min(cnt0)

    def pack_body(q, carry):
        for iref, vref in ((ti1, tv1), (ti2, tv2)):
            x = iref[pl.ds(q * 16, 16)]
            v = vref[pl.ds(q * 16, 16)]
            bn = lax.shift_right_logical(x, BIN_SHIFT)
            cnt, _last = plsc.scan_count(bn)
            cur = plsc.load_gather(offs, [bn])
            slot = cur + (cnt - rank_base)
            plsc.store_scatter(pack_i, [slot], x)
            plsc.store_scatter(pack_v, [slot], v)
            plsc.addupdate_scatter(offs, [bn], ones16)
        return carry

    lax.fori_loop(0, nv, pack_body, jnp.int32(0), unroll=2)

    pltpu.sync_copy(pack_i, bi_out.at[pl.ds(pl.multiple_of(wid * PACKW, 8), PACKW)])
    pltpu.sync_copy(pack_v, bv_out.at[pl.ds(pl.multiple_of(wid * PACKW, 8), PACKW)])
    pltpu.sync_copy(cnts, st_out.at[pl.ds(pl.multiple_of(wid * NB, 8), NB)])


@functools.partial(
    pl.kernel,
    mesh=_mesh,
    compiler_params=pltpu.CompilerParams(needs_layout_passes=False),
    out_type=jax.ShapeDtypeStruct((M,), jnp.float32),
    scratch_types=[
        pltpu.VMEM((NT * NB + 16,), jnp.int32),
        pltpu.VMEM((BSZ,), jnp.float32),
        pltpu.VMEM((BSZ,), jnp.float32),
        pltpu.VMEM((CH,), jnp.int32),
        pltpu.VMEM((CH,), jnp.float32),
        pltpu.VMEM((CH,), jnp.int32),
        pltpu.VMEM((CH,), jnp.float32),
        pltpu.VMEM((CH,), jnp.int32),
        pltpu.VMEM((CH,), jnp.float32),
        pltpu.VMEM((CH,), jnp.int32),
        pltpu.VMEM((CH,), jnp.float32),
        pltpu.VMEM((CH,), jnp.int32),
        pltpu.VMEM((CH,), jnp.float32),
        pltpu.VMEM((NT,), jnp.int32),
        pltpu.VMEM((NT,), jnp.int32),
        pltpu.VMEM((NT,), jnp.int32),
        pltpu.VMEM((NT,), jnp.int32),
        pltpu.SemaphoreType.DMA,
        pltpu.SemaphoreType.DMA,
        pltpu.SemaphoreType.DMA,
        pltpu.SemaphoreType.DMA,
        pltpu.SemaphoreType.DMA,
    ],
)
def _merge_kernel(mem, bi, bv, st, out, starts, absmax, accum,
                  b0i, b0v, b1i, b1v, b2i, b2v, b3i, b3v, bci, bcv,
                  seg_s, seg_e, seg_a0, seg_n,
                  sem0, sem1, semc, semza, semzb):
    wid = lax.axis_index("c") * NS + lax.axis_index("s")
    pltpu.sync_copy(st, starts.at[pl.ds(0, NT * NB)])
    iota16 = lax.broadcasted_iota(jnp.int32, (16,), 0)

    def sget(ref, i):
        return jnp.max(plsc.load_gather(ref, [lax.broadcast(i, (16,))]))

    def issue(tt, dbi, dbv, sem):
        a0 = sget(seg_a0, tt)
        src = pl.multiple_of(tt * PACKW + a0, 8)
        pltpu.async_copy(bi.at[pl.ds(src, CH)], dbi, sem)
        pltpu.async_copy(bv.at[pl.ds(src, CH)], dbv, sem)

    def waitb(dbi, dbv, sem):
        pltpu.make_async_copy(bi.at[pl.ds(0, CH)], dbi, sem).wait()
        pltpu.make_async_copy(bv.at[pl.ds(0, CH)], dbv, sem).wait()

    def jbody(j, jcarry):
        b = wid * BPT + j
        bin_base = pl.multiple_of(b * BSZ, 8)
        zsrc = pl.multiple_of(wid * (2 * BSZ), 8)
        cza = pltpu.async_copy(mem.at[pl.ds(zsrc, BSZ)], absmax, semza)
        czb = pltpu.async_copy(mem.at[pl.ds(zsrc + BSZ, BSZ)], accum, semzb)

        for g in range(NT // 16):
            tv = g * 16 + iota16
            idxv = tv * NB + b
            sv = plsc.load_gather(starts, [idxv])
            ev = plsc.load_gather(starts, [idxv + 1])
            ev = jnp.where(b == NB - 1, jnp.int32(PAIRS), ev)
            a0v = sv & jnp.int32(-8)
            nv_ = jnp.where(
                ev > sv,
                lax.shift_right_logical(ev - a0v + jnp.int32(CH - 1), CHS),
                0,
            )
            seg_s[pl.ds(g * 16, 16)] = sv
            seg_e[pl.ds(g * 16, 16)] = ev
            seg_a0[pl.ds(g * 16, 16)] = a0v
            seg_n[pl.ds(g * 16, 16)] = nv_

        def qloop(phase, dbi, dbv, q_lo, q_hi):
            def q_body(q, qc):
                x = dbi[pl.ds(q * 16, 16)]
                v = dbv[pl.ds(q * 16, 16)]
                bn = lax.shift_right_logical(x, BIN_SHIFT)
                msk = bn == b
                ol = x & BIN_MASK
                av = jnp.abs(v)
                if phase == 0:
                    g_ = plsc.load_gather(absmax, [ol], mask=msk)
                    m = jnp.maximum(g_, av)
                    plsc.store_scatter(absmax, [ol], m, mask=msk)
                    g2 = plsc.load_gather(absmax, [ol], mask=msk)
                    pend = jnp.where(msk & (av > g2), 1, 0)

                    def w_cond(pc):
                        return jnp.max(pc) > 0

                    def w_body(pc):
                        pm = pc > 0
                        gg = plsc.load_gather(absmax, [ol], mask=pm)
                        mm = jnp.maximum(gg, av)
                        plsc.store_scatter(absmax, [ol], mm, mask=pm)
                        gg2 = plsc.load_gather(absmax, [ol], mask=pm)
                        return jnp.where(pm & (av > gg2), 1, 0)

                    lax.while_loop(w_cond, w_body, pend)
                else:
                    g_ = plsc.load_gather(absmax, [ol], mask=msk)
                    win = msk & (av == g_)
                    plsc.addupdate_scatter(accum, [ol], v, mask=win)
                return qc

            lax.fori_loop(q_lo, q_hi, q_body, jnp.int32(0))

        def process(phase, tt, dbi, dbv):
            s = sget(seg_s, tt)
            e = sget(seg_e, tt)
            a0 = sget(seg_a0, tt)
            n = sget(seg_n, tt)
            q_lo = lax.shift_right_logical(s - a0, 4)
            q_hi = lax.shift_right_logical(jnp.minimum(e, a0 + CH) - a0 + 15, 4)
            qloop(phase, dbi, dbv, q_lo, q_hi)

            def r_body(r, rc):
                off = pl.multiple_of(a0 + r * CH, 8)
                src = pl.multiple_of(tt * PACKW + off, 8)
                c1 = pltpu.async_copy(bi.at[pl.ds(src, CH)], bci, semc)
                c2 = pltpu.async_copy(bv.at[pl.ds(src, CH)], bcv, semc)
                c1.wait()
                c2.wait()
                qh = lax.shift_right_logical(jnp.minimum(e, off + CH) - off + 15, 4)
                qloop(phase, bci, bcv, jnp.int32(0), qh)
                return rc

            lax.fori_loop(1, n, r_body, jnp.int32(0))

        def sweep(phase):
            bufs = ((b0i, b0v), (b1i, b1v), (b2i, b2v), (b3i, b3v))

            def k_body(k, kc):
                t0 = 4 * k
                for u, (dbi, dbv) in enumerate(bufs):
                    issue(t0 + u, dbi, dbv, sem0)
                for dbi, dbv in bufs:
                    waitb(dbi, dbv, sem0)
                for u, (dbi, dbv) in enumerate(bufs):
                    process(phase, t0 + u, dbi, dbv)
                return kc

            lax.fori_loop(0, NT // 4, k_body, jnp.int32(0))

        cza.wait()
        sweep(0)
        czb.wait()
        sweep(1)
        pltpu.sync_copy(accum, out.at[pl.ds(bin_base, BSZ)])
        return jcarry

    lax.fori_loop(0, BPT, jbody, jnp.int32(0))


def kernel(mem, idx1, vals1, idx2, vals2):
    pad = PADN - N
    ip = jnp.zeros((pad,), jnp.int32)
    vp = jnp.zeros((pad,), jnp.float32)
    i1 = jnp.concatenate([idx1, ip])
    v1 = jnp.concatenate([vals1, vp])
    i2 = jnp.concatenate([idx2, ip])
    v2 = jnp.concatenate([vals2, vp])
    bi, bv, st = _bin_kernel(i1, v1, i2, v2)
    return _merge_kernel(mem, bi, bv, st)


# speedup vs baseline: 87.2171x; 1.0654x over previous
import functools

import jax
import jax.numpy as jnp
from jax import lax
from jax.experimental import pallas as pl
from jax.experimental.pallas import tpu as pltpu
from jax.experimental.pallas import tpu_sc as plsc

M = 4194304
N = 419430
NC = 2
NS = 16
NT = NC * NS
CHUNK = 13120
PADN = NT * CHUNK
PAIRS = 2 * CHUNK
SLACK = 512
PACKW = PAIRS + SLACK
NB = 128
BSZ = M // NB
BIN_SHIFT = 15
BIN_MASK = BSZ - 1
BPT = NB // NT
CH = 256
CHS = CH.bit_length() - 1

_mesh = plsc.VectorSubcoreMesh(core_axis_name="c", subcore_axis_name="s")


@functools.partial(
    pl.kernel,
    mesh=_mesh,
    compiler_params=pltpu.CompilerParams(needs_layout_passes=False),
    out_type=(
        jax.ShapeDtypeStruct((NT * PACKW,), jnp.int32),
        jax.ShapeDtypeStruct((NT * PACKW,), jnp.float32),
        jax.ShapeDtypeStruct((NT * NB,), jnp.int32),
    ),
    scratch_types=[
        pltpu.VMEM((CHUNK,), jnp.int32),
        pltpu.VMEM((CHUNK,), jnp.float32),
        pltpu.VMEM((CHUNK,), jnp.int32),
        pltpu.VMEM((CHUNK,), jnp.float32),
        pltpu.VMEM((NB,), jnp.int32),
        pltpu.VMEM((NB,), jnp.int32),
        pltpu.VMEM((PACKW,), jnp.int32),
        pltpu.VMEM((PACKW,), jnp.float32),
        pltpu.SemaphoreType.DMA,
    ],
)
def _bin_kernel(i1, v1, i2, v2, bi_out, bv_out, st_out,
                ti1, tv1, ti2, tv2, cnts, offs, pack_i, pack_v, sem):
    wid = lax.axis_index("c") * NS + lax.axis_index("s")
    base = pl.multiple_of(wid * CHUNK, 8)
    cp1 = pltpu.async_copy(i1.at[pl.ds(base, CHUNK)], ti1, sem)
    cp2 = pltpu.async_copy(v1.at[pl.ds(base, CHUNK)], tv1, sem)
    cp3 = pltpu.async_copy(i2.at[pl.ds(base, CHUNK)], ti2, sem)
    cp4 = pltpu.async_copy(v2.at[pl.ds(base, CHUNK)], tv2, sem)

    zeros16 = jnp.zeros((16,), jnp.int32)
    ones16 = jnp.ones((16,), jnp.int32)
    neg16 = jnp.full((16,), -1, jnp.int32)
    for j in range(NB // 16):
        cnts[pl.ds(16 * j, 16)] = zeros16
    for j in range(SLACK // 16):
        pack_i[pl.ds(PAIRS + 16 * j, 16)] = neg16

    cp1.wait()
    cp2.wait()
    cp3.wait()
    cp4.wait()

    nv = CHUNK // 16

    def hist_body(q, carry):
        for ref in (ti1, ti2):
            x = ref[pl.ds(q * 16, 16)]
            bn = lax.shift_right_logical(x, BIN_SHIFT)
            plsc.addupdate_scatter(cnts, [bn], ones16)
        return carry

    lax.fori_loop(0, nv, hist_body, jnp.int32(0), unroll=4)

    def scan_body(j, carry):
        v = cnts[pl.ds(16 * j, 16)]
        cs = plsc.cumsum(v)
        excl = cs - v + carry
        cnts[pl.ds(16 * j, 16)] = excl
        offs[pl.ds(16 * j, 16)] = excl
        return carry + jnp.max(cs)

    lax.fori_loop(0, NB // 16, scan_body, jnp.int32(0))

    cnt0, _unused = plsc.scan_count(zeros16)
    rank_base = jnp.min(cnt0)

    def pack_body(q, carry):
        for iref, vref in ((ti1, tv1), (ti2, tv2)):
            x = iref[pl.ds(q * 16, 16)]
            v = vref[pl.ds(q * 16, 16)]
            bn = lax.shift_right_logical(x, BIN_SHIFT)
            cnt, _last = plsc.scan_count(bn)
            cur = plsc.load_gather(offs, [bn])
            slot = cur + (cnt - rank_base)
            plsc.store_scatter(pack_i, [slot], x)
            plsc.store_scatter(pack_v, [slot], v)
            plsc.addupdate_scatter(offs, [bn], ones16)
        return carry

    lax.fori_loop(0, nv, pack_body, jnp.int32(0), unroll=4)

    pltpu.sync_copy(pack_i, bi_out.at[pl.ds(pl.multiple_of(wid * PACKW, 8), PACKW)])
    pltpu.sync_copy(pack_v, bv_out.at[pl.ds(pl.multiple_of(wid * PACKW, 8), PACKW)])
    pltpu.sync_copy(cnts, st_out.at[pl.ds(pl.multiple_of(wid * NB, 8), NB)])


@functools.partial(
    pl.kernel,
    mesh=_mesh,
    compiler_params=pltpu.CompilerParams(needs_layout_passes=False),
    out_type=jax.ShapeDtypeStruct((M,), jnp.float32),
    scratch_types=[
        pltpu.VMEM((NT * NB + 16,), jnp.int32),
        pltpu.VMEM((BSZ,), jnp.float32),
        pltpu.VMEM((BSZ,), jnp.float32),
        pltpu.VMEM((CH,), jnp.int32),
        pltpu.VMEM((CH,), jnp.float32),
        pltpu.VMEM((CH,), jnp.int32),
        pltpu.VMEM((CH,), jnp.float32),
        pltpu.VMEM((CH,), jnp.int32),
        pltpu.VMEM((CH,), jnp.float32),
        pltpu.VMEM((CH,), jnp.int32),
        pltpu.VMEM((CH,), jnp.float32),
        pltpu.VMEM((CH,), jnp.int32),
        pltpu.VMEM((CH,), jnp.float32),
        pltpu.VMEM((CH,), jnp.int32),
        pltpu.VMEM((CH,), jnp.float32),
        pltpu.VMEM((CH,), jnp.int32),
        pltpu.VMEM((CH,), jnp.float32),
        pltpu.VMEM((CH,), jnp.int32),
        pltpu.VMEM((CH,), jnp.float32),
        pltpu.VMEM((CH,), jnp.int32),
        pltpu.VMEM((CH,), jnp.float32),
        pltpu.VMEM((NT,), jnp.int32),
        pltpu.VMEM((NT,), jnp.int32),
        pltpu.VMEM((NT,), jnp.int32),
        pltpu.VMEM((NT,), jnp.int32),
        pltpu.SemaphoreType.DMA,
        pltpu.SemaphoreType.DMA,
        pltpu.SemaphoreType.DMA,
        pltpu.SemaphoreType.DMA,
        pltpu.SemaphoreType.DMA,
    ],
)
def _merge_kernel(mem, bi, bv, st, out, starts, absmax, accum,
                  b0i, b0v, b1i, b1v, b2i, b2v, b3i, b3v,
                  b4i, b4v, b5i, b5v, b6i, b6v, b7i, b7v, bci, bcv,
                  seg_s, seg_e, seg_a0, seg_n,
                  sem0, sem1, semc, semza, semzb):
    wid = lax.axis_index("c") * NS + lax.axis_index("s")
    pltpu.sync_copy(st, starts.at[pl.ds(0, NT * NB)])
    iota16 = lax.broadcasted_iota(jnp.int32, (16,), 0)

    def sget(ref, i):
        return jnp.max(plsc.load_gather(ref, [lax.broadcast(i, (16,))]))

    def issue(tt, dbi, dbv, sem):
        a0 = sget(seg_a0, tt)
        src = pl.multiple_of(tt * PACKW + a0, 8)
        pltpu.async_copy(bi.at[pl.ds(src, CH)], dbi, sem)
        pltpu.async_copy(bv.at[pl.ds(src, CH)], dbv, sem)

    def waitb(dbi, dbv, sem):
        pltpu.make_async_copy(bi.at[pl.ds(0, CH)], dbi, sem).wait()
        pltpu.make_async_copy(bv.at[pl.ds(0, CH)], dbv, sem).wait()

    def jbody(j, jcarry):
        b = wid * BPT + j
        bin_base = pl.multiple_of(b * BSZ, 8)
        zsrc = pl.multiple_of(wid * (2 * BSZ), 8)
        cza = pltpu.async_copy(mem.at[pl.ds(zsrc, BSZ)], absmax, semza)
        czb = pltpu.async_copy(mem.at[pl.ds(zsrc + BSZ, BSZ)], accum, semzb)

        for g in range(NT // 16):
            tv = g * 16 + iota16
            idxv = tv * NB + b
            sv = plsc.load_gather(starts, [idxv])
            ev = plsc.load_gather(starts, [idxv + 1])
            ev = jnp.where(b == NB - 1, jnp.int32(PAIRS), ev)
            a0v = sv & jnp.int32(-8)
            nv_ = jnp.where(
                ev > sv,
                lax.shift_right_logical(ev - a0v + jnp.int32(CH - 1), CHS),
                0,
            )
            seg_s[pl.ds(g * 16, 16)] = sv
            seg_e[pl.ds(g * 16, 16)] = ev
            seg_a0[pl.ds(g * 16, 16)] = a0v
            seg_n[pl.ds(g * 16, 16)] = nv_

        def qloop(phase, dbi, dbv, q_lo, q_hi):
            def q_body(q, qc):
                x = dbi[pl.ds(q * 16, 16)]
                v = dbv[pl.ds(q * 16, 16)]
                bn = lax.shift_right_logical(x, BIN_SHIFT)
                msk = bn == b
                ol = x & BIN_MASK
                av = jnp.abs(v)
                if phase == 0:
                    g_ = plsc.load_gather(absmax, [ol], mask=msk)
                    m = jnp.maximum(g_, av)
                    plsc.store_scatter(absmax, [ol], m, mask=msk)
                    g2 = plsc.load_gather(absmax, [ol], mask=msk)
                    pend = jnp.where(msk & (av > g2), 1, 0)

                    def w_cond(pc):
                        return jnp.max(pc) > 0

                    def w_body(pc):
                        pm = pc > 0
                        gg = plsc.load_gather(absmax, [ol], mask=pm)
                        mm = jnp.maximum(gg, av)
                        plsc.store_scatter(absmax, [ol], mm, mask=pm)
                        gg2 = plsc.load_gather(absmax, [ol], mask=pm)
                        return jnp.where(pm & (av > gg2), 1, 0)

                    lax.while_loop(w_cond, w_body, pend)
                else:
                    g_ = plsc.load_gather(absmax, [ol], mask=msk)
                    win = msk & (av == g_)
                    plsc.addupdate_scatter(accum, [ol], v, mask=win)
                return qc

            lax.fori_loop(q_lo, q_hi, q_body, jnp.int32(0))

        def process(phase, tt, dbi, dbv):
            s = sget(seg_s, tt)
            e = sget(seg_e, tt)
            a0 = sget(seg_a0, tt)
            n = sget(seg_n, tt)
            q_lo = lax.shift_right_logical(s - a0, 4)
            q_hi = lax.shift_right_logical(jnp.minimum(e, a0 + CH) - a0 + 15, 4)
            qloop(phase, dbi, dbv, q_lo, q_hi)

            def r_body(r, rc):
                off = pl.multiple_of(a0 + r * CH, 8)
                src = pl.multiple_of(tt * PACKW + off, 8)
                c1 = pltpu.async_copy(bi.at[pl.ds(src, CH)], bci, semc)
                c2 = pltpu.async_copy(bv.at[pl.ds(src, CH)], bcv, semc)
                c1.wait()
                c2.wait()
                qh = lax.shift_right_logical(jnp.minimum(e, off + CH) - off + 15, 4)
                qloop(phase, bci, bcv, jnp.int32(0), qh)
                return rc

            lax.fori_loop(1, n, r_body, jnp.int32(0))

        def sweep(phase):
            bufs = ((b0i, b0v), (b1i, b1v), (b2i, b2v), (b3i, b3v),
                    (b4i, b4v), (b5i, b5v), (b6i, b6v), (b7i, b7v))

            def k_body(k, kc):
                t0 = 8 * k
                for u, (dbi, dbv) in enumerate(bufs):
                    issue(t0 + u, dbi, dbv, sem0)
                for dbi, dbv in bufs:
                    waitb(dbi, dbv, sem0)
                for u, (dbi, dbv) in enumerate(bufs):
                    process(phase, t0 + u, dbi, dbv)
                return kc

            lax.fori_loop(0, NT // 8, k_body, jnp.int32(0))

        cza.wait()
        sweep(0)
        czb.wait()
        sweep(1)
        pltpu.sync_copy(accum, out.at[pl.ds(bin_base, BSZ)])
        return jcarry

    lax.fori_loop(0, BPT, jbody, jnp.int32(0))


def kernel(mem, idx1, vals1, idx2, vals2):
    pad = PADN - N
    ip = jnp.zeros((pad,), jnp.int32)
    vp = jnp.zeros((pad,), jnp.float32)
    i1 = jnp.concatenate([idx1, ip])
    v1 = jnp.concatenate([vals1, vp])
    i2 = jnp.concatenate([idx2, ip])
    v2 = jnp.concatenate([vals2, vp])
    bi, bv, st = _bin_kernel(i1, v1, i2, v2)
    return _merge_kernel(mem, bi, bv, st)


# speedup vs baseline: 95.9729x; 1.1004x over previous
import functools

import jax
import jax.numpy as jnp
from jax import lax
from jax.experimental import pallas as pl
from jax.experimental.pallas import tpu as pltpu
from jax.experimental.pallas import tpu_sc as plsc

M = 4194304
N = 419430
NC = 2
NS = 16
NT = NC * NS
CHUNK = 13120
PADN = NT * CHUNK
PAIRS = 2 * CHUNK
SLACK = 512
PACKW = PAIRS + SLACK
NB = 128
BSZ = M // NB
BIN_SHIFT = 15
BIN_MASK = BSZ - 1
BPT = NB // NT
CH = 256
CHS = CH.bit_length() - 1

_mesh = plsc.VectorSubcoreMesh(core_axis_name="c", subcore_axis_name="s")


@functools.partial(
    pl.kernel,
    mesh=_mesh,
    compiler_params=pltpu.CompilerParams(needs_layout_passes=False),
    out_type=(
        jax.ShapeDtypeStruct((NT * PACKW,), jnp.int32),
        jax.ShapeDtypeStruct((NT * PACKW,), jnp.float32),
        jax.ShapeDtypeStruct((NT * NB,), jnp.int32),
    ),
    scratch_types=[
        pltpu.VMEM((CHUNK,), jnp.int32),
        pltpu.VMEM((CHUNK,), jnp.float32),
        pltpu.VMEM((CHUNK,), jnp.int32),
        pltpu.VMEM((CHUNK,), jnp.float32),
        pltpu.VMEM((NB,), jnp.int32),
        pltpu.VMEM((NB,), jnp.int32),
        pltpu.VMEM((PACKW,), jnp.int32),
        pltpu.VMEM((PACKW,), jnp.float32),
        pltpu.SemaphoreType.DMA,
    ],
)
def _bin_kernel(i1, v1, i2, v2, bi_out, bv_out, st_out,
                ti1, tv1, ti2, tv2, cnts, offs, pack_i, pack_v, sem):
    wid = lax.axis_index("c") * NS + lax.axis_index("s")
    base = pl.multiple_of(wid * CHUNK, 8)
    cp1 = pltpu.async_copy(i1.at[pl.ds(base, CHUNK)], ti1, sem)
    cp2 = pltpu.async_copy(v1.at[pl.ds(base, CHUNK)], tv1, sem)
    cp3 = pltpu.async_copy(i2.at[pl.ds(base, CHUNK)], ti2, sem)
    cp4 = pltpu.async_copy(v2.at[pl.ds(base, CHUNK)], tv2, sem)

    zeros16 = jnp.zeros((16,), jnp.int32)
    ones16 = jnp.ones((16,), jnp.int32)
    neg16 = jnp.full((16,), -1, jnp.int32)
    for j in range(NB // 16):
        cnts[pl.ds(16 * j, 16)] = zeros16
    for j in range(SLACK // 16):
        pack_i[pl.ds(PAIRS + 16 * j, 16)] = neg16

    cp1.wait()
    cp2.wait()
    cp3.wait()
    cp4.wait()

    nv = CHUNK // 16

    def hist_body(q, carry):
        for ref in (ti1, ti2):
            x = ref[pl.ds(q * 16, 16)]
            bn = lax.shift_right_logical(x, BIN_SHIFT)
            plsc.addupdate_scatter(cnts, [bn], ones16)
        return carry

    lax.fori_loop(0, nv, hist_body, jnp.int32(0), unroll=4)

    def scan_body(j, carry):
        v = cnts[pl.ds(16 * j, 16)]
        cs = plsc.cumsum(v)
        excl = cs - v + carry
        cnts[pl.ds(16 * j, 16)] = excl
        offs[pl.ds(16 * j, 16)] = excl
        return carry + jnp.max(cs)

    lax.fori_loop(0, NB // 16, scan_body, jnp.int32(0))

    cnt0, _unused = plsc.scan_count(zeros16)
    rank_base = jnp.min(cnt0)

    def pack_body(q, carry):
        for iref, vref in ((ti1, tv1), (ti2, tv2)):
            x = iref[pl.ds(q * 16, 16)]
            v = vref[pl.ds(q * 16, 16)]
            bn = lax.shift_right_logical(x, BIN_SHIFT)
            cnt, _last = plsc.scan_count(bn)
            cur = plsc.load_gather(offs, [bn])
            slot = cur + (cnt - rank_base)
            plsc.store_scatter(pack_i, [slot], x)
            plsc.store_scatter(pack_v, [slot], v)
            plsc.addupdate_scatter(offs, [bn], ones16)
        return carry

    lax.fori_loop(0, nv, pack_body, jnp.int32(0), unroll=4)

    pltpu.sync_copy(pack_i, bi_out.at[pl.ds(pl.multiple_of(wid * PACKW, 8), PACKW)])
    pltpu.sync_copy(pack_v, bv_out.at[pl.ds(pl.multiple_of(wid * PACKW, 8), PACKW)])
    pltpu.sync_copy(cnts, st_out.at[pl.ds(pl.multiple_of(wid * NB, 8), NB)])


@functools.partial(
    pl.kernel,
    mesh=_mesh,
    compiler_params=pltpu.CompilerParams(needs_layout_passes=False),
    out_type=jax.ShapeDtypeStruct((M,), jnp.float32),
    scratch_types=[
        pltpu.VMEM((NT * NB + 16,), jnp.int32),
        pltpu.VMEM((BSZ,), jnp.float32),
        pltpu.VMEM((BSZ,), jnp.float32),
        pltpu.VMEM((CH,), jnp.int32),
        pltpu.VMEM((CH,), jnp.float32),
        pltpu.VMEM((CH,), jnp.int32),
        pltpu.VMEM((CH,), jnp.float32),
        pltpu.VMEM((CH,), jnp.int32),
        pltpu.VMEM((CH,), jnp.float32),
        pltpu.VMEM((CH,), jnp.int32),
        pltpu.VMEM((CH,), jnp.float32),
        pltpu.VMEM((CH,), jnp.int32),
        pltpu.VMEM((CH,), jnp.float32),
        pltpu.VMEM((CH,), jnp.int32),
        pltpu.VMEM((CH,), jnp.float32),
        pltpu.VMEM((CH,), jnp.int32),
        pltpu.VMEM((CH,), jnp.float32),
        pltpu.VMEM((CH,), jnp.int32),
        pltpu.VMEM((CH,), jnp.float32),
        pltpu.VMEM((CH,), jnp.int32),
        pltpu.VMEM((CH,), jnp.float32),
        pltpu.VMEM((NT,), jnp.int32),
        pltpu.VMEM((NT,), jnp.int32),
        pltpu.VMEM((NT,), jnp.int32),
        pltpu.VMEM((NT,), jnp.int32),
        pltpu.VMEM((27600,), jnp.int32),
        pltpu.VMEM((27600,), jnp.float32),
        pltpu.SemaphoreType.DMA,
        pltpu.SemaphoreType.DMA,
        pltpu.SemaphoreType.DMA,
        pltpu.SemaphoreType.DMA,
        pltpu.SemaphoreType.DMA,
    ],
)
def _merge_kernel(mem, bi, bv, st, out, starts, absmax, accum,
                  b0i, b0v, b1i, b1v, b2i, b2v, b3i, b3v,
                  b4i, b4v, b5i, b5v, b6i, b6v, b7i, b7v, bci, bcv,
                  seg_s, seg_e, seg_a0, seg_n, cache_i, cache_v,
                  sem0, sem1, semc, semza, semzb):
    wid = lax.axis_index("c") * NS + lax.axis_index("s")
    pltpu.sync_copy(st, starts.at[pl.ds(0, NT * NB)])
    iota16 = lax.broadcasted_iota(jnp.int32, (16,), 0)

    def sget(ref, i):
        return jnp.max(plsc.load_gather(ref, [lax.broadcast(i, (16,))]))

    def issue(tt, dbi, dbv, sem):
        a0 = sget(seg_a0, tt)
        src = pl.multiple_of(tt * PACKW + a0, 8)
        pltpu.async_copy(bi.at[pl.ds(src, CH)], dbi, sem)
        pltpu.async_copy(bv.at[pl.ds(src, CH)], dbv, sem)

    def waitb(dbi, dbv, sem):
        pltpu.make_async_copy(bi.at[pl.ds(0, CH)], dbi, sem).wait()
        pltpu.make_async_copy(bv.at[pl.ds(0, CH)], dbv, sem).wait()

    def jbody(j, jcarry):
        b = wid * BPT + j
        bin_base = pl.multiple_of(b * BSZ, 8)
        zsrc = pl.multiple_of(wid * (2 * BSZ), 8)
        cza = pltpu.async_copy(mem.at[pl.ds(zsrc, BSZ)], absmax, semza)
        czb = pltpu.async_copy(mem.at[pl.ds(zsrc + BSZ, BSZ)], accum, semzb)

        for g in range(NT // 16):
            tv = g * 16 + iota16
            idxv = tv * NB + b
            sv = plsc.load_gather(starts, [idxv])
            ev = plsc.load_gather(starts, [idxv + 1])
            ev = jnp.where(b == NB - 1, jnp.int32(PAIRS), ev)
            a0v = sv & jnp.int32(-8)
            nv_ = jnp.where(
                ev > sv,
                lax.shift_right_logical(ev - a0v + jnp.int32(CH - 1), CHS),
                0,
            )
            seg_s[pl.ds(g * 16, 16)] = sv
            seg_e[pl.ds(g * 16, 16)] = ev
            seg_a0[pl.ds(g * 16, 16)] = a0v
            seg_n[pl.ds(g * 16, 16)] = nv_

        CAP = 27600

        def phase1_vreg(x, v):
            bn = lax.shift_right_logical(x, BIN_SHIFT)
            msk = bn == b
            ol = x & BIN_MASK
            av = jnp.abs(v)
            g_ = plsc.load_gather(absmax, [ol], mask=msk)
            win = msk & (av == g_)
            plsc.addupdate_scatter(accum, [ol], v, mask=win)

        def qloop(phase, dbi, dbv, q_lo, q_hi, pos):
            def q_body(q, qc):
                x = dbi[pl.ds(q * 16, 16)]
                v = dbv[pl.ds(q * 16, 16)]
                bn = lax.shift_right_logical(x, BIN_SHIFT)
                msk = bn == b
                ol = x & BIN_MASK
                av = jnp.abs(v)
                if phase == 0:
                    g_ = plsc.load_gather(absmax, [ol], mask=msk)
                    m = jnp.maximum(g_, av)
                    plsc.store_scatter(absmax, [ol], m, mask=msk)
                    g2 = plsc.load_gather(absmax, [ol], mask=msk)
                    pend = jnp.where(msk & (av > g2), 1, 0)

                    def w_cond(pc):
                        return jnp.max(pc) > 0

                    def w_body(pc):
                        pm = pc > 0
                        gg = plsc.load_gather(absmax, [ol], mask=pm)
                        mm = jnp.maximum(gg, av)
                        plsc.store_scatter(absmax, [ol], mm, mask=pm)
                        gg2 = plsc.load_gather(absmax, [ol], mask=pm)
                        return jnp.where(pm & (av > gg2), 1, 0)

                    lax.while_loop(w_cond, w_body, pend)
                    pc_ = pl.multiple_of(jnp.minimum(qc, CAP - 16) & -16, 8)
                    cache_i[pl.ds(pc_, 16)] = x
                    cache_v[pl.ds(pc_, 16)] = v
                    return qc + 16
                else:
                    g_ = plsc.load_gather(absmax, [ol], mask=msk)
                    win = msk & (av == g_)
                    plsc.addupdate_scatter(accum, [ol], v, mask=win)
                    return qc

            return lax.fori_loop(q_lo, q_hi, q_body, pos)

        def process(phase, tt, dbi, dbv, pos):
            s = sget(seg_s, tt)
            e = sget(seg_e, tt)
            a0 = sget(seg_a0, tt)
            n = sget(seg_n, tt)
            q_lo = lax.shift_right_logical(s - a0, 4)
            q_hi = lax.shift_right_logical(jnp.minimum(e, a0 + CH) - a0 + 15, 4)
            pos = qloop(phase, dbi, dbv, q_lo, q_hi, pos)

            def r_body(r, rc):
                off = pl.multiple_of(a0 + r * CH, 8)
                src = pl.multiple_of(tt * PACKW + off, 8)
                c1 = pltpu.async_copy(bi.at[pl.ds(src, CH)], bci, semc)
                c2 = pltpu.async_copy(bv.at[pl.ds(src, CH)], bcv, semc)
                c1.wait()
                c2.wait()
                qh = lax.shift_right_logical(jnp.minimum(e, off + CH) - off + 15, 4)
                return qloop(phase, bci, bcv, jnp.int32(0), qh, rc)

            return lax.fori_loop(1, n, r_body, pos)

        def sweep(phase):
            bufs = ((b0i, b0v), (b1i, b1v), (b2i, b2v), (b3i, b3v),
                    (b4i, b4v), (b5i, b5v), (b6i, b6v), (b7i, b7v))

            def k_body(k, kc):
                t0 = 8 * k
                for u, (dbi, dbv) in enumerate(bufs):
                    issue(t0 + u, dbi, dbv, sem0)
                for dbi, dbv in bufs:
                    waitb(dbi, dbv, sem0)
                for u, (dbi, dbv) in enumerate(bufs):
                    kc = process(phase, t0 + u, dbi, dbv, kc)
                return kc

            return lax.fori_loop(0, NT // 8, k_body, jnp.int32(0))

        cza.wait()
        pos_end = sweep(0)
        czb.wait()

        @pl.when(pos_end <= CAP)
        def _replay_from_cache():
            def c_body(q, qc):
                x = cache_i[pl.ds(q * 16, 16)]
                v = cache_v[pl.ds(q * 16, 16)]
                phase1_vreg(x, v)
                return qc

            lax.fori_loop(0, lax.shift_right_logical(pos_end, 4), c_body,
                          jnp.int32(0))

        @pl.when(pos_end > CAP)
        def _replay_from_hbm():
            sweep(1)

        pltpu.sync_copy(accum, out.at[pl.ds(bin_base, BSZ)])
        return jcarry

    lax.fori_loop(0, BPT, jbody, jnp.int32(0))


def kernel(mem, idx1, vals1, idx2, vals2):
    pad = PADN - N
    ip = jnp.zeros((pad,), jnp.int32)
    vp = jnp.zeros((pad,), jnp.float32)
    i1 = jnp.concatenate([idx1, ip])
    v1 = jnp.concatenate([vals1, vp])
    i2 = jnp.concatenate([idx2, ip])
    v2 = jnp.concatenate([vals2, vp])
    bi, bv, st = _bin_kernel(i1, v1, i2, v2)
    return _merge_kernel(mem, bi, bv, st)
